# baseline (speedup 1.0000x reference)
import functools

import jax
import jax.numpy as jnp
import numpy as np
from jax import lax
from jax.experimental import pallas as pl
from jax.experimental.pallas import tpu as pltpu

_ROT_A = (13, 15, 26, 6)
_ROT_B = (17, 29, 16, 24)
_KS = (np.uint32(0), np.uint32(1), np.uint32(0x1BD11BDB))
_INJ = ((1, 2), (2, 0), (0, 1), (1, 2), (2, 0))


def _rotl(x, r):
    return (x << np.uint32(r)) | (x >> np.uint32(32 - r))


def _threefry_bits(i_u32):
    x0 = jnp.zeros_like(i_u32)
    x1 = i_u32 + _KS[1]
    for g, rots in enumerate((_ROT_A, _ROT_B, _ROT_A, _ROT_B, _ROT_A)):
        for r in rots:
            x0 = x0 + x1
            x1 = _rotl(x1, r)
            x1 = x1 ^ x0
        a, b = _INJ[g]
        x0 = x0 + _KS[a]
        x1 = x1 + (_KS[b] + np.uint32(g + 1))
    return x0 ^ x1


def _sample_block(mask_ref, out_ref, vmax_ref, *, ncols, blk_c):
    c = pl.program_id(0)
    nrows = mask_ref.shape[0]
    shape = (nrows, blk_c)

    jglob = c * blk_c + lax.broadcasted_iota(jnp.int32, shape, 1)
    flat = lax.broadcasted_iota(jnp.int32, shape, 0) * ncols + jglob
    bits = _threefry_bits(flat.astype(jnp.uint32))

    f = lax.bitcast_convert_type((bits >> np.uint32(9)) | np.uint32(0x3F800000),
                                 jnp.float32) - np.float32(1.0)
    u = jnp.maximum(np.float32(1e-9), f + np.float32(1e-9))
    gumbel = -jnp.log(-jnp.log(u))
    val = jnp.log(mask_ref[...] + np.float32(1e-20)) + gumbel
    val = jnp.where(jglob < ncols, val, -jnp.inf)

    m = jnp.max(val, axis=1)
    jsel = jnp.where(val == m[:, None], jglob, np.int32(0x7FFFFFFF))
    bidx = jnp.min(jsel, axis=1)

    @pl.when(c == 0)
    def _init():
        vmax_ref[...] = m
        out_ref[...] = bidx

    @pl.when(c != 0)
    def _accum():
        cur = vmax_ref[...]
        take = m > cur
        vmax_ref[...] = jnp.where(take, m, cur)
        out_ref[...] = jnp.where(take, bidx, out_ref[...])


@jax.jit
def kernel(action_mask):
    nrows, ncols = action_mask.shape
    blk_c = 512
    grid = (pl.cdiv(ncols, blk_c),)
    return pl.pallas_call(
        functools.partial(_sample_block, ncols=ncols, blk_c=blk_c),
        grid=grid,
        in_specs=[pl.BlockSpec((nrows, blk_c), lambda c: (0, c))],
        out_specs=pl.BlockSpec((nrows,), lambda c: (0,)),
        out_shape=jax.ShapeDtypeStruct((nrows,), jnp.int32),
        scratch_shapes=[pltpu.VMEM((nrows,), jnp.float32)],
        compiler_params=pltpu.CompilerParams(
            dimension_semantics=("arbitrary",)),
    )(action_mask)
